# baseline (speedup 1.0000x reference)
import functools

import jax
import jax.numpy as jnp
from jax import lax
from jax.experimental import pallas as pl
from jax.experimental.pallas import tpu as pltpu
from jax.experimental.pallas import tpu_sc as plsc

_QB = 1024
_KB = 4352


def _psq_body(p_ref, out_ref):
    p = p_ref[...]
    out_ref[...] = jnp.sum(p * p, axis=1)[None, :]


def _argmin_body(nk, kdim, q_ref, p_ref, psq_ref, idx_ref,
                 minval_ref, minidx_ref):
    k = pl.program_id(1)

    @pl.when(k == 0)
    def _init():
        minval_ref[...] = jnp.full(minval_ref.shape, jnp.inf, jnp.float32)
        minidx_ref[...] = jnp.zeros(minidx_ref.shape, jnp.int32)

    q = q_ref[...]
    p = p_ref[...]
    cross = lax.dot_general(q, p, (((1,), (1,)), ((), ())),
                            preferred_element_type=jnp.float32)
    q_sq = jnp.sum(q * q, axis=1, keepdims=True)
    d2 = q_sq - 2.0 * cross + psq_ref[...]

    def _update(dd):
        m = jnp.min(dd, axis=1, keepdims=True)
        colf = lax.broadcasted_iota(jnp.int32, dd.shape, 1).astype(jnp.float32)
        lidx_f = jnp.min(jnp.where(dd == m, colf, jnp.float32(3.0e38)),
                         axis=1, keepdims=True)
        lidx = lidx_f.astype(jnp.int32) + k * _KB
        better = m < minval_ref[...]
        minidx_ref[...] = jnp.where(better, lidx, minidx_ref[...])
        newval = jnp.where(better, m, minval_ref[...])
        minval_ref[...] = newval.astype(jnp.bfloat16).astype(jnp.float32)

    @pl.when(k < nk - 1)
    def _full():
        _update(d2)

    @pl.when(k == nk - 1)
    def _tail():
        col = lax.broadcasted_iota(jnp.int32, d2.shape, 1)
        _update(jnp.where(col < kdim - (nk - 1) * _KB, d2, jnp.inf))
        idx_ref[...] = minidx_ref[...]


def _nn_argmin(points_q, points):
    Q, D = points_q.shape
    K = points.shape[0]
    nk = pl.cdiv(K, _KB)
    k_pad = nk * _KB
    p_sq = pl.pallas_call(
        _psq_body,
        grid=(nk,),
        in_specs=[pl.BlockSpec((_KB, D), lambda k: (k, 0))],
        out_specs=pl.BlockSpec((1, _KB), lambda k: (0, k)),
        out_shape=jax.ShapeDtypeStruct((1, k_pad), jnp.float32),
    )(points)
    idx2d = pl.pallas_call(
        functools.partial(_argmin_body, nk, K),
        grid=(Q // _QB, nk),
        in_specs=[
            pl.BlockSpec((_QB, D), lambda i, k: (i, 0)),
            pl.BlockSpec((_KB, D), lambda i, k: (k, 0)),
            pl.BlockSpec((1, _KB), lambda i, k: (0, k)),
        ],
        out_specs=pl.BlockSpec((_QB, 1), lambda i, k: (i, 0)),
        out_shape=jax.ShapeDtypeStruct((Q, 1), jnp.int32),
        scratch_shapes=[
            pltpu.VMEM((_QB, 1), jnp.float32),
            pltpu.VMEM((_QB, 1), jnp.int32),
        ],
    )(points_q, points, p_sq)
    return idx2d[:, 0]


def _gather_sc(values, idx):
    info = plsc.get_sparse_core_info()
    n_workers = info.num_cores * info.num_subcores
    B = idx.shape[0]
    D = values.shape[1]
    b_per_w = B // n_workers
    mesh = plsc.VectorSubcoreMesh(core_axis_name="c", subcore_axis_name="s")

    @functools.partial(
        pl.kernel, mesh=mesh,
        out_type=jax.ShapeDtypeStruct((B, D), jnp.float32),
        compiler_params=pltpu.CompilerParams(use_tc_tiling_on_sc=False),
        scratch_types=[
            pltpu.VMEM((b_per_w,), jnp.int32),
            pltpu.VMEM((b_per_w, D), jnp.float32),
            pltpu.SemaphoreType.DMA,
        ],
    )
    def gk(values_hbm, idx_hbm, out_hbm, idx_v, rows_v, sem):
        wid = lax.axis_index("s") * info.num_cores + lax.axis_index("c")
        base = wid * b_per_w
        pltpu.sync_copy(idx_hbm.at[pl.ds(base, b_per_w)], idx_v)
        pltpu.async_copy(values_hbm.at[idx_v], rows_v, sem).wait()
        pltpu.sync_copy(rows_v, out_hbm.at[pl.ds(base, b_per_w)])

    return gk(values, idx)


def kernel(points_q, points, values):
    nn_idx = _nn_argmin(points_q, points)
    return _gather_sc(values, nn_idx)


# speedup vs baseline: 1.0509x; 1.0509x over previous
import functools

import jax
import jax.numpy as jnp
from jax import lax
from jax.experimental import pallas as pl
from jax.experimental.pallas import tpu as pltpu
from jax.experimental.pallas import tpu_sc as plsc

_QB = 1024
_KB = 4352


def _psq_body(p_ref, out_ref):
    p = p_ref[...]
    out_ref[...] = jnp.sum(p * p, axis=1)[None, :]


def _argmin_body(nk, q_ref, p_ref, psq_ref, idx_ref, minval_ref, minidx_ref):
    k = pl.program_id(1)

    @pl.when(k == 0)
    def _init():
        minval_ref[...] = jnp.full(minval_ref.shape, jnp.inf, jnp.float32)
        minidx_ref[...] = jnp.zeros(minidx_ref.shape, jnp.int32)

    q = q_ref[...]
    p = p_ref[...]
    cross = lax.dot_general(q, p, (((1,), (1,)), ((), ())),
                            preferred_element_type=jnp.float32)
    q_sq = jnp.sum(q * q, axis=1, keepdims=True)
    d2 = q_sq - 2.0 * cross + psq_ref[...]
    m = jnp.min(d2, axis=1, keepdims=True)
    colf = lax.broadcasted_iota(jnp.int32, d2.shape, 1).astype(jnp.float32)
    lidx_f = jnp.min(jnp.where(d2 == m, colf, jnp.float32(3.0e38)),
                     axis=1, keepdims=True)
    lidx = lidx_f.astype(jnp.int32) + k * _KB
    better = m < minval_ref[...]
    minidx_ref[...] = jnp.where(better, lidx, minidx_ref[...])
    newval = jnp.where(better, m, minval_ref[...])
    minval_ref[...] = newval.astype(jnp.bfloat16).astype(jnp.float32)

    @pl.when(k == nk - 1)
    def _emit():
        idx_ref[...] = minidx_ref[...]


def _nn_argmin(points_q, points):
    Q, D = points_q.shape
    K = points.shape[0]
    nk = pl.cdiv(K, _KB)
    k_pad = nk * _KB
    points_p = jnp.pad(points, ((0, k_pad - K), (0, 0)),
                       constant_values=1.0e20)
    p_sq = pl.pallas_call(
        _psq_body,
        grid=(nk,),
        in_specs=[pl.BlockSpec((_KB, D), lambda k: (k, 0))],
        out_specs=pl.BlockSpec((1, _KB), lambda k: (0, k)),
        out_shape=jax.ShapeDtypeStruct((1, k_pad), jnp.float32),
    )(points_p)
    idx2d = pl.pallas_call(
        functools.partial(_argmin_body, nk),
        grid=(Q // _QB, nk),
        in_specs=[
            pl.BlockSpec((_QB, D), lambda i, k: (i, 0)),
            pl.BlockSpec((_KB, D), lambda i, k: (k, 0)),
            pl.BlockSpec((1, _KB), lambda i, k: (0, k)),
        ],
        out_specs=pl.BlockSpec((_QB, 1), lambda i, k: (i, 0)),
        out_shape=jax.ShapeDtypeStruct((Q, 1), jnp.int32),
        scratch_shapes=[
            pltpu.VMEM((_QB, 1), jnp.float32),
            pltpu.VMEM((_QB, 1), jnp.int32),
        ],
    )(points_q, points_p, p_sq)
    return idx2d[:, 0]


def _gather_sc(values, idx):
    info = plsc.get_sparse_core_info()
    n_workers = info.num_cores * info.num_subcores
    B = idx.shape[0]
    D = values.shape[1]
    b_per_w = B // n_workers
    mesh = plsc.VectorSubcoreMesh(core_axis_name="c", subcore_axis_name="s")

    @functools.partial(
        pl.kernel, mesh=mesh,
        out_type=jax.ShapeDtypeStruct((B, D), jnp.float32),
        compiler_params=pltpu.CompilerParams(use_tc_tiling_on_sc=False),
        scratch_types=[
            pltpu.VMEM((b_per_w,), jnp.int32),
            pltpu.VMEM((b_per_w, D), jnp.float32),
            pltpu.SemaphoreType.DMA,
        ],
    )
    def gk(values_hbm, idx_hbm, out_hbm, idx_v, rows_v, sem):
        wid = lax.axis_index("s") * info.num_cores + lax.axis_index("c")
        base = wid * b_per_w
        pltpu.sync_copy(idx_hbm.at[pl.ds(base, b_per_w)], idx_v)
        pltpu.async_copy(values_hbm.at[idx_v], rows_v, sem).wait()
        pltpu.sync_copy(rows_v, out_hbm.at[pl.ds(base, b_per_w)])

    return gk(values, idx)


def kernel(points_q, points, values):
    nn_idx = _nn_argmin(points_q, points)
    return _gather_sc(values, nn_idx)
